# baseline (speedup 1.0000x reference)
import jax
import jax.numpy as jnp
from jax.experimental import pallas as pl

PATCH = 2
N_PRED = 4
SLIDING = 3
NODE = 512
IN_DIM = 16
HID = 64
N_HIST = 12
B = 8

F32 = jnp.float32


def _dot(a, b):
    return jax.lax.dot(a, b, preferred_element_type=F32)


def _gdot(a, b):
    return jax.lax.dot(a, b, preferred_element_type=F32)


def _to_fm(q, d):
    return jnp.concatenate([q[b * d:(b + 1) * d, :] for b in range(B)], axis=1)


def _to_q(fm, d):
    n = NODE
    return jnp.concatenate([fm[:, b * n:(b + 1) * n] for b in range(B)],
                           axis=0)


def _glu(lin, out):
    return lin[:out] * jax.nn.sigmoid(lin[out:2 * out])


def _cell_conv_first(graphT, xs_q, d, out, MT, bias):
    T = len(xs_q)
    Bd = B * d
    Bn = B * NODE
    Zall = jnp.concatenate(xs_q, axis=0) if T > 1 else xs_q[0]
    SZ = _gdot(Zall, graphT)
    CTall = jnp.concatenate(
        [jnp.concatenate([_to_fm(SZ[t * Bd:(t + 1) * Bd, :], d),
                          _to_fm(xs_q[t], d)], axis=0)
         for t in range(T)], axis=1)
    Pall = _dot(MT, CTall)
    P = [Pall[:, t * Bn:(t + 1) * Bn] for t in range(T)]
    outs = []
    for t in range(T):
        lin = P[t][:2 * out] + bias
        if t + 1 < T:
            lin = lin + P[t + 1][2 * out:]
        outs.append(_glu(lin, out))
    return outs


def _cell_proj_first(graphT, hs, out, K, bias):
    T = len(hs)
    o2 = 2 * out
    Bn = B * NODE
    PRall = _dot(K, jnp.concatenate(hs, axis=1))
    PR = [PRall[:, t * Bn:(t + 1) * Bn] for t in range(T)]
    linS = []
    linX = []
    for t in range(T):
        s = PR[t][:o2]
        x = PR[t][2 * o2:3 * o2]
        if t + 1 < T:
            s = s + PR[t + 1][o2:2 * o2]
            x = x + PR[t + 1][3 * o2:]
        linS.append(s)
        linX.append(x)
    Gq = _gdot(jnp.concatenate([_to_q(s, o2) for s in linS], axis=0), graphT)
    Bo2 = B * o2
    outs = []
    for t in range(T):
        g = _to_fm(Gq[t * Bo2:(t + 1) * Bo2, :], o2)
        outs.append(_glu(g + linX[t] + bias, out))
    return outs


def _forward_kernel(xq_ref, nv2T_ref, nv1T_ref, gdT_ref,
                    lt0_M, lt0_b, lt1_K, lt1_b,
                    st0_M, st0_b, st1_K, st1_b,
                    wa_ref, ba_ref, out_ref):
    graphT = _dot(nv2T_ref[...], nv1T_ref[...]) + gdT_ref[...]
    Bd = B * IN_DIM
    xq = xq_ref[...]
    xs_q = [xq[t * Bd:(t + 1) * Bd, :] for t in range(N_HIST)]

    h0 = _cell_conv_first(graphT, xs_q, IN_DIM, HID, lt0_M[...], lt0_b[...])
    lt = _cell_proj_first(graphT, h0, IN_DIM, lt1_K[...], lt1_b[...])

    wa = wa_ref[...]
    ba = ba_ref[...]
    M0, b0 = st0_M[...], st0_b[...]
    K1, b1 = st1_K[...], st1_b[...]

    def score_exp(q_fm):
        sc = jnp.sum(q_fm * wa, axis=0, keepdims=True) + ba
        return jnp.exp(jnp.tanh(sc))

    den_lt = None
    acc_lt = None
    for q in lt:
        e = score_exp(q)
        den_lt = e if den_lt is None else den_lt + e
        w = e * q
        acc_lt = w if acc_lt is None else acc_lt + w

    xs_fm = {}
    P0 = {}

    Bn = B * NODE
    h_frames = list(range(N_HIST - SLIDING, N_HIST))
    sz_h = _gdot(jnp.concatenate([xs_q[t] for t in h_frames], axis=0), graphT)
    CT_h = jnp.concatenate(
        [jnp.concatenate([_to_fm(sz_h[k * Bd:(k + 1) * Bd, :], IN_DIM),
                          _to_fm(xs_q[t], IN_DIM)], axis=0)
         for k, t in enumerate(h_frames)], axis=1)
    P0_h = _dot(M0, CT_h)
    for k, t in enumerate(h_frames):
        P0[t] = P0_h[:, k * Bn:(k + 1) * Bn]

    def proj0(t):
        if t not in P0:
            if t < N_HIST:
                sz = _gdot(xs_q[t], graphT)
                bot = _to_fm(xs_q[t], IN_DIM)
            else:
                sz = _gdot(_to_q(xs_fm[t], IN_DIM), graphT)
                bot = xs_fm[t]
            CT = jnp.concatenate([_to_fm(sz, IN_DIM), bot], axis=0)
            P0[t] = _dot(M0, CT)
        return P0[t]

    W0 = {}

    def win0(i):
        if i not in W0:
            lin = proj0(i)[:2 * HID] + proj0(i + 1)[2 * HID:] + b0
            W0[i] = _glu(lin, HID)
        return W0[i]

    PR1 = {}

    def proj1(i):
        if i not in PR1:
            PR1[i] = _dot(K1, win0(i))
        return PR1[i]

    o2 = 2 * IN_DIM
    Bo2 = B * o2
    for p in range(N_PRED):
        f0 = N_HIST - SLIDING + p
        cpad = _glu(proj0(f0 + 2)[:2 * HID] + b0, HID)
        prc = _dot(K1, cpad)
        pa, pb = proj1(f0), proj1(f0 + 1)
        linS_a = pa[:o2] + pb[o2:2 * o2]
        linS_b = pb[:o2] + prc[o2:2 * o2]
        linS_c = prc[:o2]
        Gq = _gdot(jnp.concatenate(
            [_to_q(linS_a, o2), _to_q(linS_b, o2), _to_q(linS_c, o2)],
            axis=0), graphT)
        sh = [
            _glu(_to_fm(Gq[:Bo2], o2)
                 + pa[2 * o2:3 * o2] + pb[3 * o2:] + b1, IN_DIM),
            _glu(_to_fm(Gq[Bo2:2 * Bo2], o2)
                 + pb[2 * o2:3 * o2] + prc[3 * o2:] + b1, IN_DIM),
            _glu(_to_fm(Gq[2 * Bo2:], o2)
                 + prc[2 * o2:3 * o2] + b1, IN_DIM),
        ]
        den = den_lt
        acc = acc_lt
        for q in sh:
            e = score_exp(q)
            den = den + e
            acc = acc + e * q
        o = acc * (1.0 / den)
        for b in range(B):
            out_ref[b * (N_PRED * NODE) + p * NODE:
                    b * (N_PRED * NODE) + (p + 1) * NODE, :] = (
                jnp.transpose(o[:, b * NODE:(b + 1) * NODE]))
        xs_fm[N_HIST + p] = o


def _pack_cell(W1, b1, W2, b2):
    d = W1.shape[0] // PATCH
    out = W2.shape[1]
    W2ext = jnp.concatenate([W2, jnp.zeros_like(W2)], axis=1)
    M1 = jnp.concatenate([W1[:d], W2ext[:d]], axis=0)
    M2 = jnp.concatenate([W1[d:], W2ext[d:]], axis=0)
    MT = jnp.concatenate([M1, M2], axis=1).T
    bias = (b1 + jnp.concatenate([b2, jnp.zeros_like(b2)], axis=0))[:, None]
    return MT, bias


def _pack_cell_pf(W1, b1, W2, b2):
    MT, bias = _pack_cell(W1, b1, W2, b2)
    d = W1.shape[0] // PATCH
    o2 = MT.shape[0] // 2
    K = jnp.concatenate([MT[:o2, :d], MT[o2:, :d],
                         MT[:o2, d:], MT[o2:, d:]], axis=0)
    return K, bias


def kernel(x, targets, batch_seen, nodevec1, nodevec2, graph_dense,
           lt0_W1, lt0_b1, lt0_W2, lt0_b2, lt1_W1, lt1_b1, lt1_W2, lt1_b2,
           st0_W1, st0_b1, st0_W2, st0_b2, st1_W1, st1_b1, st1_W2, st1_b2,
           att_W, att_b):
    xq = jnp.transpose(x, (1, 0, 3, 2)).reshape(N_HIST * B * IN_DIM, NODE)
    lt0 = _pack_cell(lt0_W1, lt0_b1, lt0_W2, lt0_b2)
    lt1 = _pack_cell_pf(lt1_W1, lt1_b1, lt1_W2, lt1_b2)
    st0 = _pack_cell(st0_W1, st0_b1, st0_W2, st0_b2)
    st1 = _pack_cell_pf(st1_W1, st1_b1, st1_W2, st1_b2)
    wa = att_W.reshape(IN_DIM, 1)
    ba = att_b.reshape(1, 1)

    outf = pl.pallas_call(
        _forward_kernel,
        out_shape=jax.ShapeDtypeStruct((B * N_PRED * NODE, IN_DIM), F32),
    )(xq, nodevec2.T, nodevec1.T, graph_dense.T, *lt0, *lt1, *st0, *st1,
      wa, ba)

    return outf.reshape(B, N_PRED, NODE, IN_DIM)


# speedup vs baseline: 1.1033x; 1.1033x over previous
import jax
import jax.numpy as jnp
from jax.experimental import pallas as pl

PATCH = 2
N_PRED = 4
SLIDING = 3
NODE = 512
IN_DIM = 16
HID = 64
N_HIST = 12
B = 8

F32 = jnp.float32


def _dot(a, b):
    return jax.lax.dot(a, b, preferred_element_type=F32)


def _gdot(a, b):
    return jax.lax.dot(a, b, preferred_element_type=F32)


def _to_fm(q, d):
    return jnp.concatenate([q[b * d:(b + 1) * d, :] for b in range(B)], axis=1)


def _to_q(fm, d):
    n = NODE
    return jnp.concatenate([fm[:, b * n:(b + 1) * n] for b in range(B)],
                           axis=0)


def _glu(lin, out):
    return lin[:out] * jax.nn.sigmoid(lin[out:2 * out])


def _cell_conv_first(graphT, xs_q, d, out, MT, bias):
    T = len(xs_q)
    Bd = B * d
    Bn = B * NODE
    Zall = jnp.concatenate(xs_q, axis=0) if T > 1 else xs_q[0]
    SZ = _gdot(Zall, graphT)
    CTall = jnp.concatenate(
        [jnp.concatenate([_to_fm(SZ[t * Bd:(t + 1) * Bd, :], d),
                          _to_fm(xs_q[t], d)], axis=0)
         for t in range(T)], axis=1)
    Pall = _dot(MT, CTall)
    P = [Pall[:, t * Bn:(t + 1) * Bn] for t in range(T)]
    outs = []
    for t in range(T):
        lin = P[t][:2 * out] + bias
        if t + 1 < T:
            lin = lin + P[t + 1][2 * out:]
        outs.append(_glu(lin, out))
    return outs


def _cell_proj_first(graphT, hs, out, K, bias):
    T = len(hs)
    o2 = 2 * out
    Bn = B * NODE
    PRall = _dot(K, jnp.concatenate(hs, axis=1))
    PR = [PRall[:, t * Bn:(t + 1) * Bn] for t in range(T)]
    linS = []
    linX = []
    for t in range(T):
        s = PR[t][:o2]
        x = PR[t][2 * o2:3 * o2]
        if t + 1 < T:
            s = s + PR[t + 1][o2:2 * o2]
            x = x + PR[t + 1][3 * o2:]
        linS.append(s)
        linX.append(x)
    Gq = _gdot(jnp.concatenate([_to_q(s, o2) for s in linS], axis=0), graphT)
    Bo2 = B * o2
    outs = []
    for t in range(T):
        g = _to_fm(Gq[t * Bo2:(t + 1) * Bo2, :], o2)
        outs.append(_glu(g + linX[t] + bias, out))
    return outs


def _forward_kernel(xq_ref, nv2T_ref, nv1T_ref, gdT_ref,
                    lt0_M, lt0_b, lt1_K, lt1_b,
                    st0_M, st0_b, st1_K, st1_b,
                    wa_ref, ba_ref, out_ref):
    graphT = _dot(nv2T_ref[...], nv1T_ref[...]) + gdT_ref[...]
    Bd = B * IN_DIM
    xq = xq_ref[...]
    xs_q = [xq[t * Bd:(t + 1) * Bd, :] for t in range(N_HIST)]

    h0 = _cell_conv_first(graphT, xs_q, IN_DIM, HID, lt0_M[...], lt0_b[...])
    lt = _cell_proj_first(graphT, h0, IN_DIM, lt1_K[...], lt1_b[...])

    wa = wa_ref[...]
    ba = ba_ref[...]
    M0, b0 = st0_M[...], st0_b[...]
    K1, b1 = st1_K[...], st1_b[...]

    def score_exp(q_fm):
        sc = jnp.sum(q_fm * wa, axis=0, keepdims=True) + ba
        return jnp.exp(jnp.tanh(sc))

    den_lt = None
    acc_lt = None
    for q in lt:
        e = score_exp(q)
        den_lt = e if den_lt is None else den_lt + e
        w = e * q
        acc_lt = w if acc_lt is None else acc_lt + w

    xs_fm = {}
    P0 = {}

    Bn = B * NODE
    h_frames = list(range(N_HIST - SLIDING, N_HIST))
    sz_h = _gdot(jnp.concatenate([xs_q[t] for t in h_frames], axis=0), graphT)
    CT_h = jnp.concatenate(
        [jnp.concatenate([_to_fm(sz_h[k * Bd:(k + 1) * Bd, :], IN_DIM),
                          _to_fm(xs_q[t], IN_DIM)], axis=0)
         for k, t in enumerate(h_frames)], axis=1)
    P0_h = _dot(M0, CT_h)
    for k, t in enumerate(h_frames):
        P0[t] = P0_h[:, k * Bn:(k + 1) * Bn]

    def proj0(t):
        if t not in P0:
            if t < N_HIST:
                sz = _gdot(xs_q[t], graphT)
                bot = _to_fm(xs_q[t], IN_DIM)
            else:
                sz = _gdot(_to_q(xs_fm[t], IN_DIM), graphT)
                bot = xs_fm[t]
            CT = jnp.concatenate([_to_fm(sz, IN_DIM), bot], axis=0)
            P0[t] = _dot(M0, CT)
        return P0[t]

    W0 = {}

    def win0(i):
        if i not in W0:
            lin = proj0(i)[:2 * HID] + proj0(i + 1)[2 * HID:] + b0
            W0[i] = _glu(lin, HID)
        return W0[i]

    PR1 = {}

    def proj1(i):
        if i not in PR1:
            PR1[i] = _dot(K1, win0(i))
        return PR1[i]

    o2 = 2 * IN_DIM
    Bo2 = B * o2
    for p in range(N_PRED):
        f0 = N_HIST - SLIDING + p
        cpad = _glu(proj0(f0 + 2)[:2 * HID] + b0, HID)
        prc = _dot(K1, cpad)
        pa, pb = proj1(f0), proj1(f0 + 1)
        linS_a = pa[:o2] + pb[o2:2 * o2]
        linS_b = pb[:o2] + prc[o2:2 * o2]
        linS_c = prc[:o2]
        Gq = _gdot(jnp.concatenate(
            [_to_q(linS_a, o2), _to_q(linS_b, o2), _to_q(linS_c, o2)],
            axis=0), graphT)
        sh = [
            _glu(_to_fm(Gq[:Bo2], o2)
                 + pa[2 * o2:3 * o2] + pb[3 * o2:] + b1, IN_DIM),
            _glu(_to_fm(Gq[Bo2:2 * Bo2], o2)
                 + pb[2 * o2:3 * o2] + prc[3 * o2:] + b1, IN_DIM),
            _glu(_to_fm(Gq[2 * Bo2:], o2)
                 + prc[2 * o2:3 * o2] + b1, IN_DIM),
        ]
        den = den_lt
        acc = acc_lt
        for q in sh:
            e = score_exp(q)
            den = den + e
            acc = acc + e * q
        o = acc * (1.0 / den)
        out_ref[p * IN_DIM:(p + 1) * IN_DIM, :] = o
        xs_fm[N_HIST + p] = o


def _pack_cell(W1, b1, W2, b2):
    d = W1.shape[0] // PATCH
    out = W2.shape[1]
    W2ext = jnp.concatenate([W2, jnp.zeros_like(W2)], axis=1)
    M1 = jnp.concatenate([W1[:d], W2ext[:d]], axis=0)
    M2 = jnp.concatenate([W1[d:], W2ext[d:]], axis=0)
    MT = jnp.concatenate([M1, M2], axis=1).T
    bias = (b1 + jnp.concatenate([b2, jnp.zeros_like(b2)], axis=0))[:, None]
    return MT, bias


def _pack_cell_pf(W1, b1, W2, b2):
    MT, bias = _pack_cell(W1, b1, W2, b2)
    d = W1.shape[0] // PATCH
    o2 = MT.shape[0] // 2
    K = jnp.concatenate([MT[:o2, :d], MT[o2:, :d],
                         MT[:o2, d:], MT[o2:, d:]], axis=0)
    return K, bias


def kernel(x, targets, batch_seen, nodevec1, nodevec2, graph_dense,
           lt0_W1, lt0_b1, lt0_W2, lt0_b2, lt1_W1, lt1_b1, lt1_W2, lt1_b2,
           st0_W1, st0_b1, st0_W2, st0_b2, st1_W1, st1_b1, st1_W2, st1_b2,
           att_W, att_b):
    xq = jnp.transpose(x, (1, 0, 3, 2)).reshape(N_HIST * B * IN_DIM, NODE)
    lt0 = _pack_cell(lt0_W1, lt0_b1, lt0_W2, lt0_b2)
    lt1 = _pack_cell_pf(lt1_W1, lt1_b1, lt1_W2, lt1_b2)
    st0 = _pack_cell(st0_W1, st0_b1, st0_W2, st0_b2)
    st1 = _pack_cell_pf(st1_W1, st1_b1, st1_W2, st1_b2)
    wa = att_W.reshape(IN_DIM, 1)
    ba = att_b.reshape(1, 1)

    outf = pl.pallas_call(
        _forward_kernel,
        out_shape=jax.ShapeDtypeStruct((N_PRED * IN_DIM, B * NODE), F32),
    )(xq, nodevec2.T, nodevec1.T, graph_dense.T, *lt0, *lt1, *st0, *st1,
      wa, ba)

    return jnp.transpose(outf.reshape(N_PRED, IN_DIM, B, NODE), (2, 0, 3, 1))
